# baseline (device time: 23182 ns/iter reference)
import jax
import jax.numpy as jnp
from jax import lax
from jax.experimental import pallas as pl
from jax.experimental.pallas import tpu as pltpu

N_CHUNK = 8


def kernel(x):
    m_per, n = x.shape
    half_rows = m_per // 2
    rows = half_rows // N_CHUNK

    def body(x_ref, out_ref, x_send_sems, x_recv_sems, z_send_sems,
             z_recv_sems):
        my_x = lax.axis_index("x")
        my_y = lax.axis_index("y")
        my_z = lax.axis_index("z")
        x_peer = (1 - my_x, my_y, my_z)
        z_peer = (my_x, my_y, my_z ^ 1)

        half = my_z % 2
        mine = my_x * m_per
        send_base = mine + half * half_rows
        x_base = (1 - my_x) * m_per + half * half_rows
        z_base = (1 - my_x) * m_per + (1 - half) * half_rows

        barrier_sem = pltpu.get_barrier_semaphore()
        for peer in (x_peer, z_peer):
            pl.semaphore_signal(
                barrier_sem, inc=1, device_id=peer,
                device_id_type=pl.DeviceIdType.MESH,
            )
        pl.semaphore_wait(barrier_sem, 2)

        out_ref[pl.ds(send_base, half_rows), :] = (
            x_ref[pl.ds(half * half_rows, half_rows), :].astype(jnp.bfloat16)
        )
        x_rdmas = []
        for k in range(N_CHUNK):
            r = pltpu.make_async_remote_copy(
                src_ref=out_ref.at[pl.ds(send_base + k * rows, rows)],
                dst_ref=out_ref.at[pl.ds(send_base + k * rows, rows)],
                send_sem=x_send_sems.at[k],
                recv_sem=x_recv_sems.at[k],
                device_id=x_peer,
                device_id_type=pl.DeviceIdType.MESH,
            )
            r.start()
            x_rdmas.append(r)

        other = 1 - half
        out_ref[pl.ds(mine + other * half_rows, half_rows), :] = (
            x_ref[pl.ds(other * half_rows, half_rows), :].astype(jnp.bfloat16)
        )

        z_rdmas = []
        for k in range(N_CHUNK):
            x_rdmas[k].wait_recv()
            r = pltpu.make_async_remote_copy(
                src_ref=out_ref.at[pl.ds(x_base + k * rows, rows)],
                dst_ref=out_ref.at[pl.ds(x_base + k * rows, rows)],
                send_sem=z_send_sems.at[k],
                recv_sem=z_recv_sems.at[k],
                device_id=z_peer,
                device_id_type=pl.DeviceIdType.MESH,
            )
            r.start()
            z_rdmas.append(r)

        for k in range(N_CHUNK):
            recv_only = pltpu.make_async_remote_copy(
                src_ref=out_ref.at[pl.ds(z_base + k * rows, rows)],
                dst_ref=out_ref.at[pl.ds(z_base + k * rows, rows)],
                send_sem=z_send_sems.at[k],
                recv_sem=z_recv_sems.at[k],
                device_id=z_peer,
                device_id_type=pl.DeviceIdType.MESH,
            )
            recv_only.wait_recv()

        for k in range(N_CHUNK):
            x_rdmas[k].wait_send()
            z_rdmas[k].wait_send()

    return pl.pallas_call(
        body,
        out_shape=jax.ShapeDtypeStruct((2 * m_per, n), jnp.bfloat16),
        in_specs=[pl.BlockSpec(memory_space=pltpu.VMEM)],
        out_specs=pl.BlockSpec(memory_space=pltpu.VMEM),
        scratch_shapes=[
            pltpu.SemaphoreType.DMA((N_CHUNK,)),
            pltpu.SemaphoreType.DMA((N_CHUNK,)),
            pltpu.SemaphoreType.DMA((N_CHUNK,)),
            pltpu.SemaphoreType.DMA((N_CHUNK,)),
        ],
        compiler_params=pltpu.CompilerParams(collective_id=0),
    )(x)
